# Initial kernel scaffold; baseline (speedup 1.0000x reference)
import jax
import jax.numpy as jnp
from jax.experimental import pallas as pl


def kernel(latents, embedding_weight, pmf_logits):
    raise NotImplementedError("write your pallas kernel here")


# speedup vs baseline: 2.9623x; 2.9623x over previous
import math

import jax
import jax.numpy as jnp
from jax.experimental import pallas as pl
from jax.experimental.pallas import tpu as pltpu

K = 1024
D = 32
LMBDA = 0.05

TM = 512


def _vq_body(x_ref, sx2_ref, et_ref, e_ref, e2_ref, bias_ref, l2p_ref,
             q_ref, ind_ref, stats_ref):
    i = pl.program_id(0)
    x = x_ref[...]
    mm = jnp.dot(x, et_ref[...], preferred_element_type=jnp.float32)
    dist = ((sx2_ref[...] + e2_ref[...]) - 2.0 * mm) + bias_ref[...]
    dmin = jnp.min(dist, axis=1, keepdims=True)
    kio = jax.lax.broadcasted_iota(jnp.int32, (TM, K), 1)
    ind = jnp.min(jnp.where(dist == dmin, kio, K), axis=1, keepdims=True)
    oh = (kio == ind).astype(jnp.float32)
    q = jnp.dot(oh, e_ref[...], preferred_element_type=jnp.float32)
    q_ref[...] = q
    ind_ref[...] = ind
    mse_p = jnp.sum((q - x) ** 2)
    rate_p = jnp.sum(oh * l2p_ref[...])
    sio = jax.lax.broadcasted_iota(jnp.int32, (8, 128), 0)
    upd = (jnp.where(sio == 0, mse_p, 0.0)
           + jnp.where(sio == 1, rate_p, 0.0))

    @pl.when(i == 0)
    def _():
        stats_ref[...] = jnp.zeros_like(stats_ref)

    stats_ref[...] += upd


def kernel(latents, embedding_weight, pmf_logits):
    N, H, W = latents.shape
    target_rows = H % D
    if target_rows != 0:
        pad_len = D - target_rows
        latents_e = jnp.concatenate([latents, latents[:, -pad_len:, :]], axis=1)
    else:
        latents_e = latents
    Hp = latents_e.shape[1]
    flat = jnp.transpose(latents_e, (0, 2, 1)).reshape(N, W, Hp // D, D).reshape(-1, D)
    M = flat.shape[0]

    sx2 = jnp.sum(flat ** 2, axis=1, keepdims=True)
    e2 = jnp.sum(embedding_weight ** 2, axis=1)[None, :]
    log_pmf = jax.nn.log_softmax(pmf_logits)
    log2_pmf = log_pmf / -math.log(2.0)
    rate_bias = (log2_pmf / LMBDA)[None, :]
    l2p = log2_pmf[None, :]
    et = embedding_weight.T

    grid = (M // TM,)
    qf, inds, stats = pl.pallas_call(
        _vq_body,
        grid=grid,
        in_specs=[
            pl.BlockSpec((TM, D), lambda i: (i, 0)),
            pl.BlockSpec((TM, 1), lambda i: (i, 0)),
            pl.BlockSpec((D, K), lambda i: (0, 0)),
            pl.BlockSpec((K, D), lambda i: (0, 0)),
            pl.BlockSpec((1, K), lambda i: (0, 0)),
            pl.BlockSpec((1, K), lambda i: (0, 0)),
            pl.BlockSpec((1, K), lambda i: (0, 0)),
        ],
        out_specs=[
            pl.BlockSpec((TM, D), lambda i: (i, 0)),
            pl.BlockSpec((TM, 1), lambda i: (i, 0)),
            pl.BlockSpec((8, 128), lambda i: (0, 0)),
        ],
        out_shape=[
            jax.ShapeDtypeStruct((M, D), jnp.float32),
            jax.ShapeDtypeStruct((M, 1), jnp.int32),
            jax.ShapeDtypeStruct((8, 128), jnp.float32),
        ],
    )(flat, sx2, et, embedding_weight, e2, rate_bias, l2p)

    quantized = qf.reshape(N, W, Hp)
    quantized = jnp.transpose(quantized, (0, 2, 1))[:, :H, :]
    mse_loss = stats[0, 0] / jnp.float32(M * D)
    rate_uem = stats[1, 0]
    prior_dist = jnp.zeros(1, dtype=jnp.float32)
    param_bit = jnp.zeros(1, dtype=jnp.float32)
    return (quantized, mse_loss, inds, rate_uem, prior_dist, param_bit)
